# baseline (speedup 1.0000x reference)
import functools

import jax
import jax.numpy as jnp
from jax.experimental import pallas as pl
from jax.experimental.pallas import tpu as pltpu


def _fused_kernel(x_ref, w16_ref, w32_ref, gamma_ref, beta_ref, o_ref,
                  xres_ref, gram_ref, sum_ref, scale_ref, shift_ref,
                  *, nb, steps, inv_m, eps):
    nb_, c_in, hh, ww = x_ref.shape
    hw = hh * ww
    p = pl.program_id(0)
    j = pl.program_id(1)

    @pl.when(p == 0)
    def _phase0():
        @pl.when(j == 0)
        def _init():
            gram_ref[...] = jnp.zeros_like(gram_ref)
            sum_ref[...] = jnp.zeros_like(sum_ref)

        x = x_ref[...].reshape(nb, c_in, hw)
        xb = x.astype(jnp.bfloat16)
        xres_ref[pl.ds(j * nb, nb)] = xb
        g = jax.lax.dot_general(
            xb, xb,
            dimension_numbers=(((2,), (2,)), ((0,), (0,))),
            preferred_element_type=jnp.float32)
        gram_ref[...] += jnp.sum(g, axis=0)
        sum_ref[...] += jnp.sum(x, axis=(0, 2))[:, None]

        @pl.when(j == steps - 1)
        def _fold():
            w = w32_ref[...]
            mean_x = sum_ref[...] * jnp.float32(inv_m)
            a = jnp.dot(w, gram_ref[...], preferred_element_type=jnp.float32)
            ey2 = jnp.sum(a * w, axis=1, keepdims=True) * jnp.float32(inv_m)
            mean_y = jnp.dot(w, mean_x, preferred_element_type=jnp.float32)
            var = jnp.maximum(ey2 - mean_y * mean_y, 0.0)
            inv_std = jax.lax.rsqrt(var + jnp.float32(eps))
            scale = gamma_ref[...] * inv_std
            scale_ref[...] = scale
            shift_ref[...] = beta_ref[...] - mean_y * scale

    @pl.when(p == 1)
    def _phase1():
        w = w16_ref[...]
        scale = scale_ref[...]
        shift = shift_ref[...]
        c_out = o_ref.shape[1]
        for i in range(nb):
            xb = xres_ref[j * nb + i]
            y = jnp.dot(w, xb, preferred_element_type=jnp.float32)
            z = y * scale + shift
            z = jnp.where(z > 0, z, jnp.float32(0.2) * z)
            o_ref[i] = z.reshape(c_out, hh, ww)

def _conv_block1(x_nchw, weight, gamma, beta, *, eps=1e-5):
    N, C_in, H, W = x_nchw.shape
    C_out = weight.shape[0]
    HW = H * W
    M = N * HW

    w32 = weight.reshape(C_out, C_in).astype(jnp.float32)
    wb16 = w32.astype(jnp.bfloat16)
    gamma2 = gamma.reshape(C_out, 1).astype(jnp.float32)
    beta2 = beta.reshape(C_out, 1).astype(jnp.float32)

    nb = 1
    steps = N // nb

    flops_mm = 2 * M * C_in * C_out
    bytes_x = M * C_in * 4
    bytes_out = M * C_out * 4

    x_spec = pl.BlockSpec(
        (nb, C_in, H, W),
        lambda p, j, s=steps: (jnp.where(p == 0, j, s - 1), 0, 0, 0))
    o_spec = pl.BlockSpec(
        (nb, C_out, H, W),
        lambda p, j: (jnp.where(p == 0, 0, j), 0, 0, 0))
    const_spec = lambda shape: pl.BlockSpec(shape, lambda p, j: (0, 0))

    out4 = pl.pallas_call(
        functools.partial(_fused_kernel, nb=nb, steps=steps,
                          inv_m=1.0 / M, eps=float(eps)),
        out_shape=jax.ShapeDtypeStruct((N, C_out, H, W), jnp.float32),
        grid=(2, steps),
        in_specs=[x_spec,
                  const_spec((C_out, C_in)),
                  const_spec((C_out, C_in)),
                  const_spec((C_out, 1)),
                  const_spec((C_out, 1))],
        out_specs=o_spec,
        scratch_shapes=[
            pltpu.VMEM((N, C_in, HW), jnp.bfloat16),
            pltpu.VMEM((C_in, C_in), jnp.float32),
            pltpu.VMEM((C_in, 1), jnp.float32),
            pltpu.VMEM((C_out, 1), jnp.float32),
            pltpu.VMEM((C_out, 1), jnp.float32),
        ],
        compiler_params=pltpu.CompilerParams(
            dimension_semantics=("arbitrary", "arbitrary"),
            vmem_limit_bytes=100 * 1024 * 1024),
        cost_estimate=pl.CostEstimate(flops=flops_mm + M * C_in * C_in,
                                      transcendentals=0,
                                      bytes_accessed=2 * bytes_x + 2 * bytes_out),
    )(x_nchw, wb16, w32, gamma2, beta2)

    return out4


def kernel(x_nchw, weight, bias, gamma, beta):
    del bias
    return _conv_block1(x_nchw, weight, gamma, beta)


# speedup vs baseline: 6.1320x; 6.1320x over previous
import functools

import jax
import jax.numpy as jnp
from jax.experimental import pallas as pl
from jax.experimental.pallas import tpu as pltpu


def _fused_kernel(x_ref, wt16_ref, wt32_ref, gamma_ref, beta_ref, o_ref,
                  xres_ref, gram_ref, sum_ref, scale_ref, shift_ref,
                  *, mt, steps, inv_m, eps):
    p = pl.program_id(0)
    j = pl.program_id(1)

    @pl.when(p == 0)
    def _phase0():
        @pl.when(j == 0)
        def _init():
            gram_ref[...] = jnp.zeros_like(gram_ref)
            sum_ref[...] = jnp.zeros_like(sum_ref)

        x = x_ref[...]
        xb = x.astype(jnp.bfloat16)
        xres_ref[pl.ds(j * mt, mt)] = xb
        gram_ref[...] += jax.lax.dot_general(
            xb, xb,
            dimension_numbers=(((0,), (0,)), ((), ())),
            preferred_element_type=jnp.float32)
        sum_ref[...] += jnp.sum(x, axis=0, keepdims=True)

        @pl.when(j == steps - 1)
        def _fold():
            wt = wt32_ref[...]
            mean_x = sum_ref[...] * jnp.float32(inv_m)
            a = jnp.dot(gram_ref[...], wt,
                        preferred_element_type=jnp.float32)
            ey2 = jnp.sum(a * wt, axis=0, keepdims=True) * jnp.float32(inv_m)
            mean_y = jnp.dot(mean_x, wt,
                             preferred_element_type=jnp.float32)
            var = jnp.maximum(ey2 - mean_y * mean_y, 0.0)
            inv_std = jax.lax.rsqrt(var + jnp.float32(eps))
            scale = gamma_ref[...] * inv_std
            scale_ref[...] = scale
            shift_ref[...] = beta_ref[...] - mean_y * scale

    @pl.when(p == 1)
    def _phase1():
        xb = xres_ref[pl.ds(j * mt, mt)]
        y = jnp.dot(xb, wt16_ref[...],
                    preferred_element_type=jnp.float32)
        z = y * scale_ref[...] + shift_ref[...]
        o_ref[...] = jnp.where(z > 0, z, jnp.float32(0.2) * z)


def _conv_block1(x_nchw, weight, gamma, beta, *, eps=1e-5):
    N, C_in, H, W = x_nchw.shape
    C_out = weight.shape[0]
    HW = H * W
    M = N * HW

    x2 = x_nchw.transpose(0, 2, 3, 1).reshape(M, C_in)
    wt32 = weight.reshape(C_out, C_in).T.astype(jnp.float32)
    wt16 = wt32.astype(jnp.bfloat16)
    gamma2 = gamma.reshape(1, C_out).astype(jnp.float32)
    beta2 = beta.reshape(1, C_out).astype(jnp.float32)

    steps = 8
    mt = M // steps

    flops_mm = 2 * M * C_in * C_out
    bytes_x = M * C_in * 4
    bytes_out = M * C_out * 4

    x_spec = pl.BlockSpec(
        (mt, C_in), lambda p, j, s=steps: (jnp.where(p == 0, j, s - 1), 0))
    o_spec = pl.BlockSpec(
        (mt, C_out), lambda p, j: (jnp.where(p == 0, 0, j), 0))
    const_spec = lambda shape: pl.BlockSpec(shape, lambda p, j: (0, 0))

    out2 = pl.pallas_call(
        functools.partial(_fused_kernel, mt=mt, steps=steps,
                          inv_m=1.0 / M, eps=float(eps)),
        out_shape=jax.ShapeDtypeStruct((M, C_out), jnp.float32),
        grid=(2, steps),
        in_specs=[x_spec,
                  const_spec((C_in, C_out)),
                  const_spec((C_in, C_out)),
                  const_spec((1, C_out)),
                  const_spec((1, C_out))],
        out_specs=o_spec,
        scratch_shapes=[
            pltpu.VMEM((M, C_in), jnp.bfloat16),
            pltpu.VMEM((C_in, C_in), jnp.float32),
            pltpu.VMEM((1, C_in), jnp.float32),
            pltpu.VMEM((1, C_out), jnp.float32),
            pltpu.VMEM((1, C_out), jnp.float32),
        ],
        compiler_params=pltpu.CompilerParams(
            dimension_semantics=("arbitrary", "arbitrary"),
            vmem_limit_bytes=100 * 1024 * 1024),
        cost_estimate=pl.CostEstimate(flops=flops_mm + M * C_in * C_in,
                                      transcendentals=0,
                                      bytes_accessed=bytes_x + bytes_out),
    )(x2, wt16, wt32, gamma2, beta2)

    return out2.reshape(N, H, W, C_out).transpose(0, 3, 1, 2)


def kernel(x_nchw, weight, bias, gamma, beta):
    del bias
    return _conv_block1(x_nchw, weight, gamma, beta)


# speedup vs baseline: 6.3707x; 1.0389x over previous
import functools

import jax
import jax.numpy as jnp
from jax.experimental import pallas as pl
from jax.experimental.pallas import tpu as pltpu


def _fused_kernel(x_hbm, wt16_ref, wt32_ref, gamma_ref, beta_ref, o_ref,
                  xv_ref, gram_ref, sum_ref, scale_ref, shift_ref, sems,
                  *, mt, steps, inv_m, eps):
    p = pl.program_id(0)
    j = pl.program_id(1)

    def chunk_copy(k):
        return pltpu.make_async_copy(
            x_hbm.at[pl.ds(k * mt, mt)], xv_ref.at[pl.ds(k * mt, mt)],
            sems.at[k])

    @pl.when(p == 0)
    def _phase0():
        @pl.when(j == 0)
        def _init():
            gram_ref[...] = jnp.zeros_like(gram_ref)
            sum_ref[...] = jnp.zeros_like(sum_ref)
            for k in range(steps):
                chunk_copy(k).start()

        chunk_copy(j).wait()
        x = xv_ref[pl.ds(j * mt, mt)]
        xb = x.astype(jnp.bfloat16)
        gram_ref[...] += jax.lax.dot_general(
            xb, xb,
            dimension_numbers=(((0,), (0,)), ((), ())),
            preferred_element_type=jnp.float32)
        sum_ref[...] += jnp.sum(x, axis=0, keepdims=True)

        @pl.when(j == steps - 1)
        def _fold():
            wt = wt32_ref[...]
            mean_x = sum_ref[...] * jnp.float32(inv_m)
            a = jnp.dot(gram_ref[...], wt,
                        preferred_element_type=jnp.float32)
            ey2 = jnp.sum(a * wt, axis=0, keepdims=True) * jnp.float32(inv_m)
            mean_y = jnp.dot(mean_x, wt,
                             preferred_element_type=jnp.float32)
            var = jnp.maximum(ey2 - mean_y * mean_y, 0.0)
            inv_std = jax.lax.rsqrt(var + jnp.float32(eps))
            scale = gamma_ref[...] * inv_std
            scale_ref[...] = scale
            shift_ref[...] = beta_ref[...] - mean_y * scale

    @pl.when(p == 1)
    def _phase1():
        xb = xv_ref[pl.ds(j * mt, mt)].astype(jnp.bfloat16)
        y = jnp.dot(xb, wt16_ref[...],
                    preferred_element_type=jnp.float32)
        z = y * scale_ref[...] + shift_ref[...]
        o_ref[...] = jnp.where(z > 0, z, jnp.float32(0.2) * z)


def _conv_block1(x_nchw, weight, gamma, beta, *, eps=1e-5):
    N, C_in, H, W = x_nchw.shape
    C_out = weight.shape[0]
    HW = H * W
    M = N * HW

    x2 = x_nchw.transpose(0, 2, 3, 1).reshape(M, C_in)
    wt32 = weight.reshape(C_out, C_in).T.astype(jnp.float32)
    wt16 = wt32.astype(jnp.bfloat16)
    gamma2 = gamma.reshape(1, C_out).astype(jnp.float32)
    beta2 = beta.reshape(1, C_out).astype(jnp.float32)

    steps = 8
    mt = M // steps

    flops_mm = 2 * M * C_in * C_out
    bytes_x = M * C_in * 4
    bytes_out = M * C_out * 4

    o_spec = pl.BlockSpec(
        (mt, C_out), lambda p, j: (jnp.where(p == 0, 0, j), 0))
    const_spec = lambda shape: pl.BlockSpec(shape, lambda p, j: (0, 0))

    out2 = pl.pallas_call(
        functools.partial(_fused_kernel, mt=mt, steps=steps,
                          inv_m=1.0 / M, eps=float(eps)),
        out_shape=jax.ShapeDtypeStruct((M, C_out), jnp.float32),
        grid=(2, steps),
        in_specs=[pl.BlockSpec(memory_space=pl.ANY),
                  const_spec((C_in, C_out)),
                  const_spec((C_in, C_out)),
                  const_spec((1, C_out)),
                  const_spec((1, C_out))],
        out_specs=o_spec,
        scratch_shapes=[
            pltpu.VMEM((M, C_in), jnp.float32),
            pltpu.VMEM((C_in, C_in), jnp.float32),
            pltpu.VMEM((1, C_in), jnp.float32),
            pltpu.VMEM((1, C_out), jnp.float32),
            pltpu.VMEM((1, C_out), jnp.float32),
            pltpu.SemaphoreType.DMA((steps,)),
        ],
        compiler_params=pltpu.CompilerParams(
            dimension_semantics=("arbitrary", "arbitrary"),
            vmem_limit_bytes=100 * 1024 * 1024),
        cost_estimate=pl.CostEstimate(flops=flops_mm + M * C_in * C_in,
                                      transcendentals=0,
                                      bytes_accessed=bytes_x + bytes_out),
    )(x2, wt16, wt32, gamma2, beta2)

    return out2.reshape(N, H, W, C_out).transpose(0, 3, 1, 2)


def kernel(x_nchw, weight, bias, gamma, beta):
    del bias
    return _conv_block1(x_nchw, weight, gamma, beta)


# speedup vs baseline: 6.4715x; 1.0158x over previous
import functools

import jax
import jax.numpy as jnp
from jax.experimental import pallas as pl
from jax.experimental.pallas import tpu as pltpu


def _fused_kernel(x_hbm, wt16_ref, wt32_ref, gamma_ref, beta_ref, o_ref,
                  xv_ref, gram_ref, sum_ref, scale_ref, shift_ref, sems,
                  *, mt, steps, inv_m, eps):
    p = pl.program_id(0)
    j = pl.program_id(1)

    def chunk_copy(k):
        return pltpu.make_async_copy(
            x_hbm.at[pl.ds(k * mt, mt)], xv_ref.at[pl.ds(k * mt, mt)],
            sems.at[k])

    @pl.when(p == 0)
    def _phase0():
        @pl.when(j == 0)
        def _init():
            gram_ref[...] = jnp.zeros_like(gram_ref)
            sum_ref[...] = jnp.zeros_like(sum_ref)
            for k in range(steps):
                chunk_copy(k).start()

        chunk_copy(j).wait()
        x = xv_ref[pl.ds(j * mt, mt)]
        xb = x.astype(jnp.bfloat16)
        xt = jnp.swapaxes(xb, 0, 1)
        gram_ref[...] += jax.lax.dot_general(
            xt, xt,
            dimension_numbers=(((1,), (1,)), ((), ())),
            preferred_element_type=jnp.float32)
        sum_ref[...] += jnp.sum(x, axis=0, keepdims=True)

        @pl.when(j == steps - 1)
        def _fold():
            wt = wt32_ref[...]
            mean_x = sum_ref[...] * jnp.float32(inv_m)
            a = jnp.dot(gram_ref[...], wt,
                        preferred_element_type=jnp.float32)
            ey2 = jnp.sum(a * wt, axis=0, keepdims=True) * jnp.float32(inv_m)
            mean_y = jnp.dot(mean_x, wt,
                             preferred_element_type=jnp.float32)
            var = jnp.maximum(ey2 - mean_y * mean_y, 0.0)
            inv_std = jax.lax.rsqrt(var + jnp.float32(eps))
            scale = gamma_ref[...] * inv_std
            scale_ref[...] = scale
            shift_ref[...] = beta_ref[...] - mean_y * scale

    @pl.when(p == 1)
    def _phase1():
        xb = xv_ref[pl.ds(j * mt, mt)].astype(jnp.bfloat16)
        y = jnp.dot(xb, wt16_ref[...],
                    preferred_element_type=jnp.float32)
        z = y * scale_ref[...] + shift_ref[...]
        o_ref[...] = jnp.where(z > 0, z, jnp.float32(0.2) * z)


def _conv_block1(x_nchw, weight, gamma, beta, *, eps=1e-5):
    N, C_in, H, W = x_nchw.shape
    C_out = weight.shape[0]
    HW = H * W
    M = N * HW

    x2 = x_nchw.transpose(0, 2, 3, 1).reshape(M, C_in)
    wt32 = weight.reshape(C_out, C_in).T.astype(jnp.float32)
    wt16 = wt32.astype(jnp.bfloat16)
    gamma2 = gamma.reshape(1, C_out).astype(jnp.float32)
    beta2 = beta.reshape(1, C_out).astype(jnp.float32)

    steps = 8
    mt = M // steps

    flops_mm = 2 * M * C_in * C_out
    bytes_x = M * C_in * 4
    bytes_out = M * C_out * 4

    o_spec = pl.BlockSpec(
        (mt, C_out), lambda p, j: (jnp.where(p == 0, 0, j), 0))
    const_spec = lambda shape: pl.BlockSpec(shape, lambda p, j: (0, 0))

    out2 = pl.pallas_call(
        functools.partial(_fused_kernel, mt=mt, steps=steps,
                          inv_m=1.0 / M, eps=float(eps)),
        out_shape=jax.ShapeDtypeStruct((M, C_out), jnp.float32),
        grid=(2, steps),
        in_specs=[pl.BlockSpec(memory_space=pl.ANY),
                  const_spec((C_in, C_out)),
                  const_spec((C_in, C_out)),
                  const_spec((1, C_out)),
                  const_spec((1, C_out))],
        out_specs=o_spec,
        scratch_shapes=[
            pltpu.VMEM((M, C_in), jnp.float32),
            pltpu.VMEM((C_in, C_in), jnp.float32),
            pltpu.VMEM((1, C_in), jnp.float32),
            pltpu.VMEM((1, C_out), jnp.float32),
            pltpu.VMEM((1, C_out), jnp.float32),
            pltpu.SemaphoreType.DMA((steps,)),
        ],
        compiler_params=pltpu.CompilerParams(
            dimension_semantics=("arbitrary", "arbitrary"),
            vmem_limit_bytes=100 * 1024 * 1024),
        cost_estimate=pl.CostEstimate(flops=flops_mm + M * C_in * C_in,
                                      transcendentals=0,
                                      bytes_accessed=bytes_x + bytes_out),
    )(x2, wt16, wt32, gamma2, beta2)

    return out2.reshape(N, H, W, C_out).transpose(0, 3, 1, 2)


def kernel(x_nchw, weight, bias, gamma, beta):
    del bias
    return _conv_block1(x_nchw, weight, gamma, beta)
